# baseline (speedup 1.0000x reference)
import functools

import jax
import jax.numpy as jnp
from jax import lax
from jax.experimental import pallas as pl
from jax.experimental.pallas import tpu as pltpu
from jax.experimental.pallas import tpu_sc as plsc

VOCAB = 50368
HIDDEN = 768
EPS = 1e-05

N_TOKENS = 4 * 8192
NSEG = 4
SEG = N_TOKENS // NSEG
NUM_CORES = 2
NUM_SUBCORES = 16
NUM_WORKERS = NUM_CORES * NUM_SUBCORES
PER_WORKER = SEG // NUM_WORKERS
CHUNK = 64
NBUF = 2
NUM_CHUNKS = PER_WORKER // CHUNK

ROW_BLK = 2048
SEG_BLKS = SEG // ROW_BLK


def _gather_body(ids_hbm, table_hbm, out_hbm, idx_all, buf_v, gsem0, gsem1,
                 wsem0, wsem1):
    wid = lax.axis_index("s") * NUM_CORES + lax.axis_index("c")
    base = wid * PER_WORKER
    gsems = (gsem0, gsem1)
    wsems = (wsem0, wsem1)

    pltpu.sync_copy(ids_hbm.at[pl.ds(base, PER_WORKER)], idx_all)

    def idx_slice(ci):
        return idx_all.at[pl.ds(pl.multiple_of(ci * CHUNK, CHUNK), CHUNK)]

    def out_slice(ci):
        return out_hbm.at[pl.ds(pl.multiple_of(base + ci * CHUNK, CHUNK), CHUNK)]

    def g_start(ci, b):
        pltpu.async_copy(table_hbm.at[idx_slice(ci)], buf_v.at[b], gsems[b])

    def g_wait(ci, b):
        pltpu.make_async_copy(table_hbm.at[idx_slice(ci)], buf_v.at[b],
                              gsems[b]).wait()

    def wb_start(ci, b):
        pltpu.async_copy(buf_v.at[b], out_slice(ci), wsems[b])

    def wb_wait(ci, b):
        pltpu.make_async_copy(buf_v.at[b], out_slice(ci), wsems[b]).wait()

    g_start(0, 0)

    def outer(g, carry):
        for b in range(NBUF):
            ci = g * NBUF + b
            nb = 1 - b
            g_wait(ci, b)
            wb_start(ci, b)

            @pl.when(ci + 1 < NUM_CHUNKS)
            def _():
                @pl.when(ci >= 1)
                def _():
                    wb_wait(ci - 1, nb)
                g_start(ci + 1, nb)
        return carry

    lax.fori_loop(0, NUM_CHUNKS // NBUF, outer, 0)

    wb_wait(NUM_CHUNKS - 2, 0)
    wb_wait(NUM_CHUNKS - 1, 1)


_sc_gather = functools.partial(
    pl.kernel,
    mesh=plsc.VectorSubcoreMesh(core_axis_name="c", subcore_axis_name="s"),
    out_type=jax.ShapeDtypeStruct((SEG, HIDDEN), jnp.float32),
    scratch_types=[
        pltpu.VMEM((PER_WORKER,), jnp.int32),
        pltpu.VMEM((NBUF, CHUNK, HIDDEN), jnp.float32),
        pltpu.SemaphoreType.DMA,
        pltpu.SemaphoreType.DMA,
        pltpu.SemaphoreType.DMA,
        pltpu.SemaphoreType.DMA,
    ],
    compiler_params=pltpu.CompilerParams(needs_layout_passes=False),
)(_gather_body)


def _ln_block(x_ref, w_ref, o_ref):
    x = x_ref[...]
    mean = jnp.mean(x, axis=1, keepdims=True)
    xc = x - mean
    var = jnp.mean(xc * xc, axis=1, keepdims=True)
    o_ref[...] = xc * lax.rsqrt(var + EPS) * w_ref[...]


def _ln_first_body(x_ref, w_ref, o_ref):
    _ln_block(x_ref, w_ref, o_ref)


def _ln_rest_body(x_ref, w_ref, prev_ref, o_ref):
    del prev_ref
    _ln_block(x_ref, w_ref, o_ref)


_ln_first = pl.pallas_call(
    _ln_first_body,
    grid=(SEG_BLKS,),
    in_specs=[
        pl.BlockSpec((ROW_BLK, HIDDEN), lambda i: (i, 0)),
        pl.BlockSpec((1, HIDDEN), lambda i: (0, 0)),
    ],
    out_specs=pl.BlockSpec((ROW_BLK, HIDDEN), lambda i: (i, 0)),
    out_shape=jax.ShapeDtypeStruct((N_TOKENS, HIDDEN), jnp.float32),
    compiler_params=pltpu.CompilerParams(
        dimension_semantics=("arbitrary",)),
)


def _make_ln_rest(seg):
    return pl.pallas_call(
        _ln_rest_body,
        grid=(SEG_BLKS,),
        in_specs=[
            pl.BlockSpec((ROW_BLK, HIDDEN), lambda i: (i, 0)),
            pl.BlockSpec((1, HIDDEN), lambda i: (0, 0)),
            pl.BlockSpec(memory_space=pltpu.MemorySpace.HBM),
        ],
        out_specs=pl.BlockSpec((ROW_BLK, HIDDEN),
                               lambda i, s=seg: (i + s * SEG_BLKS, 0)),
        out_shape=jax.ShapeDtypeStruct((N_TOKENS, HIDDEN), jnp.float32),
        input_output_aliases={2: 0},
        compiler_params=pltpu.CompilerParams(
            dimension_semantics=("arbitrary",)),
    )


_ln_rest = [_make_ln_rest(s) for s in range(1, NSEG)]


@jax.jit
def kernel(input_ids, tok_embeddings, norm_weight):
    ids = input_ids.reshape(-1).astype(jnp.int32)
    w2 = norm_weight.reshape(1, HIDDEN)
    embs = [_sc_gather(lax.slice(ids, (k * SEG,), ((k + 1) * SEG,)),
                       tok_embeddings) for k in range(NSEG)]
    out = _ln_first(embs[0], w2)
    for k in range(1, NSEG):
        out = _ln_rest[k - 1](embs[k], w2, out)
    return out.reshape(input_ids.shape + (HIDDEN,))


# speedup vs baseline: 1.0083x; 1.0083x over previous
import functools

import jax
import jax.numpy as jnp
from jax import lax
from jax.experimental import pallas as pl
from jax.experimental.pallas import tpu as pltpu
from jax.experimental.pallas import tpu_sc as plsc

VOCAB = 50368
HIDDEN = 768
EPS = 1e-05

N_TOKENS = 4 * 8192
NSEG = 2
SEG = N_TOKENS // NSEG
NUM_CORES = 2
NUM_SUBCORES = 16
NUM_WORKERS = NUM_CORES * NUM_SUBCORES
PER_WORKER = SEG // NUM_WORKERS
CHUNK = 64
NBUF = 2
NUM_CHUNKS = PER_WORKER // CHUNK

ROW_BLK = 2048
SEG_BLKS = SEG // ROW_BLK


def _gather_body(ids_hbm, table_hbm, out_hbm, idx_all, buf_v, gsem0, gsem1,
                 wsem0, wsem1):
    wid = lax.axis_index("s") * NUM_CORES + lax.axis_index("c")
    base = wid * PER_WORKER
    gsems = (gsem0, gsem1)
    wsems = (wsem0, wsem1)

    pltpu.sync_copy(ids_hbm.at[pl.ds(base, PER_WORKER)], idx_all)

    def idx_slice(ci):
        return idx_all.at[pl.ds(pl.multiple_of(ci * CHUNK, CHUNK), CHUNK)]

    def out_slice(ci):
        return out_hbm.at[pl.ds(pl.multiple_of(base + ci * CHUNK, CHUNK), CHUNK)]

    def g_start(ci, b):
        pltpu.async_copy(table_hbm.at[idx_slice(ci)], buf_v.at[b], gsems[b])

    def g_wait(ci, b):
        pltpu.make_async_copy(table_hbm.at[idx_slice(ci)], buf_v.at[b],
                              gsems[b]).wait()

    def wb_start(ci, b):
        pltpu.async_copy(buf_v.at[b], out_slice(ci), wsems[b])

    def wb_wait(ci, b):
        pltpu.make_async_copy(buf_v.at[b], out_slice(ci), wsems[b]).wait()

    g_start(0, 0)

    def outer(g, carry):
        for b in range(NBUF):
            ci = g * NBUF + b
            nb = 1 - b
            g_wait(ci, b)
            wb_start(ci, b)

            @pl.when(ci + 1 < NUM_CHUNKS)
            def _():
                @pl.when(ci >= 1)
                def _():
                    wb_wait(ci - 1, nb)
                g_start(ci + 1, nb)
        return carry

    lax.fori_loop(0, NUM_CHUNKS // NBUF, outer, 0)

    wb_wait(NUM_CHUNKS - 2, 0)
    wb_wait(NUM_CHUNKS - 1, 1)


_sc_gather = functools.partial(
    pl.kernel,
    mesh=plsc.VectorSubcoreMesh(core_axis_name="c", subcore_axis_name="s"),
    out_type=jax.ShapeDtypeStruct((SEG, HIDDEN), jnp.float32),
    scratch_types=[
        pltpu.VMEM((PER_WORKER,), jnp.int32),
        pltpu.VMEM((NBUF, CHUNK, HIDDEN), jnp.float32),
        pltpu.SemaphoreType.DMA,
        pltpu.SemaphoreType.DMA,
        pltpu.SemaphoreType.DMA,
        pltpu.SemaphoreType.DMA,
    ],
    compiler_params=pltpu.CompilerParams(needs_layout_passes=False),
)(_gather_body)


def _ln_block(x_ref, w_ref, o_ref):
    x = x_ref[...]
    mean = jnp.mean(x, axis=1, keepdims=True)
    xc = x - mean
    var = jnp.mean(xc * xc, axis=1, keepdims=True)
    o_ref[...] = xc * lax.rsqrt(var + EPS) * w_ref[...]


def _ln_first_body(x_ref, w_ref, o_ref):
    _ln_block(x_ref, w_ref, o_ref)


def _ln_rest_body(x_ref, w_ref, prev_ref, o_ref):
    del prev_ref
    _ln_block(x_ref, w_ref, o_ref)


_ln_first = pl.pallas_call(
    _ln_first_body,
    grid=(SEG_BLKS,),
    in_specs=[
        pl.BlockSpec((ROW_BLK, HIDDEN), lambda i: (i, 0)),
        pl.BlockSpec((1, HIDDEN), lambda i: (0, 0)),
    ],
    out_specs=pl.BlockSpec((ROW_BLK, HIDDEN), lambda i: (i, 0)),
    out_shape=jax.ShapeDtypeStruct((N_TOKENS, HIDDEN), jnp.float32),
    compiler_params=pltpu.CompilerParams(
        dimension_semantics=("arbitrary",)),
)


def _make_ln_rest(seg):
    return pl.pallas_call(
        _ln_rest_body,
        grid=(SEG_BLKS,),
        in_specs=[
            pl.BlockSpec((ROW_BLK, HIDDEN), lambda i: (i, 0)),
            pl.BlockSpec((1, HIDDEN), lambda i: (0, 0)),
            pl.BlockSpec(memory_space=pltpu.MemorySpace.HBM),
        ],
        out_specs=pl.BlockSpec((ROW_BLK, HIDDEN),
                               lambda i, s=seg: (i + s * SEG_BLKS, 0)),
        out_shape=jax.ShapeDtypeStruct((N_TOKENS, HIDDEN), jnp.float32),
        input_output_aliases={2: 0},
        compiler_params=pltpu.CompilerParams(
            dimension_semantics=("arbitrary",)),
    )


_ln_rest = [_make_ln_rest(s) for s in range(1, NSEG)]


@jax.jit
def kernel(input_ids, tok_embeddings, norm_weight):
    ids = input_ids.reshape(-1).astype(jnp.int32)
    w2 = norm_weight.reshape(1, HIDDEN)
    embs = [_sc_gather(lax.slice(ids, (k * SEG,), ((k + 1) * SEG,)),
                       tok_embeddings) for k in range(NSEG)]
    out = _ln_first(embs[0], w2)
    for k in range(1, NSEG):
        out = _ln_rest[k - 1](embs[k], w2, out)
    return out.reshape(input_ids.shape + (HIDDEN,))


# speedup vs baseline: 1.0485x; 1.0399x over previous
import functools

import jax
import jax.numpy as jnp
from jax import lax
from jax.experimental import pallas as pl
from jax.experimental.pallas import tpu as pltpu
from jax.experimental.pallas import tpu_sc as plsc

VOCAB = 50368
HIDDEN = 768
EPS = 1e-05

N_TOKENS = 4 * 8192
NUM_CORES = 2
NUM_SUBCORES = 16
NUM_WORKERS = NUM_CORES * NUM_SUBCORES
PER_WORKER = N_TOKENS // NUM_WORKERS
CHUNK = 32
NBUF = 4
NUM_CHUNKS = PER_WORKER // CHUNK

ROW_BLK = 2048


def _gather_body(ids_hbm, table_hbm, out_hbm, idx_all, buf_v, *sems):
    gsems = sems[:NBUF]
    wsems = sems[NBUF:]
    wid = lax.axis_index("s") * NUM_CORES + lax.axis_index("c")
    base = wid * PER_WORKER

    pltpu.sync_copy(ids_hbm.at[pl.ds(base, PER_WORKER)], idx_all)

    def idx_slice(ci):
        return idx_all.at[pl.ds(pl.multiple_of(ci * CHUNK, CHUNK), CHUNK)]

    def out_slice(ci):
        return out_hbm.at[pl.ds(pl.multiple_of(base + ci * CHUNK, CHUNK), CHUNK)]

    def g_start(ci, b):
        pltpu.async_copy(table_hbm.at[idx_slice(ci)], buf_v.at[b], gsems[b])

    def g_wait(ci, b):
        pltpu.make_async_copy(table_hbm.at[idx_slice(ci)], buf_v.at[b],
                              gsems[b]).wait()

    def wb_start(ci, b):
        pltpu.async_copy(buf_v.at[b], out_slice(ci), wsems[b])

    def wb_wait(ci, b):
        pltpu.make_async_copy(buf_v.at[b], out_slice(ci), wsems[b]).wait()

    for c0 in range(NBUF - 1):
        g_start(c0, c0)

    def outer(g, carry):
        for b0 in range(NBUF):
            ci = g * NBUF + b0
            b = b0
            g_wait(ci, b)
            wb_start(ci, b)

            cj = ci + NBUF - 1
            bj = (b0 + NBUF - 1) % NBUF

            @pl.when(cj < NUM_CHUNKS)
            def _():
                @pl.when(ci >= 1)
                def _():
                    wb_wait(ci - 1, bj)
                g_start(cj, bj)
        return carry

    lax.fori_loop(0, NUM_CHUNKS // NBUF, outer, 0)

    for k in range(NBUF):
        ci = NUM_CHUNKS - NBUF + k
        wb_wait(ci, ci % NBUF)


_sc_gather = functools.partial(
    pl.kernel,
    mesh=plsc.VectorSubcoreMesh(core_axis_name="c", subcore_axis_name="s"),
    out_type=jax.ShapeDtypeStruct((N_TOKENS, HIDDEN), jnp.float32),
    scratch_types=[
        pltpu.VMEM((PER_WORKER,), jnp.int32),
        pltpu.VMEM((NBUF, CHUNK, HIDDEN), jnp.float32),
    ] + [pltpu.SemaphoreType.DMA] * (2 * NBUF),
    compiler_params=pltpu.CompilerParams(needs_layout_passes=False),
)(_gather_body)


def _ln_body(x_ref, w_ref, o_ref):
    x = x_ref[...]
    mean = jnp.mean(x, axis=1, keepdims=True)
    xc = x - mean
    var = jnp.mean(xc * xc, axis=1, keepdims=True)
    o_ref[...] = xc * lax.rsqrt(var + EPS) * w_ref[...]


_tc_layernorm = pl.pallas_call(
    _ln_body,
    grid=(N_TOKENS // ROW_BLK,),
    in_specs=[
        pl.BlockSpec((ROW_BLK, HIDDEN), lambda i: (i, 0)),
        pl.BlockSpec((1, HIDDEN), lambda i: (0, 0)),
    ],
    out_specs=pl.BlockSpec((ROW_BLK, HIDDEN), lambda i: (i, 0)),
    out_shape=jax.ShapeDtypeStruct((N_TOKENS, HIDDEN), jnp.float32),
    compiler_params=pltpu.CompilerParams(
        dimension_semantics=("arbitrary",)),
)


@jax.jit
def kernel(input_ids, tok_embeddings, norm_weight):
    ids = input_ids.reshape(-1).astype(jnp.int32)
    emb = _sc_gather(ids, tok_embeddings)
    out = _tc_layernorm(emb, norm_weight.reshape(1, HIDDEN))
    return out.reshape(input_ids.shape + (HIDDEN,))
